# baseline (speedup 1.0000x reference)
import functools
import math

import jax
import jax.numpy as jnp
from jax import lax
from jax.experimental import pallas as pl
from jax.experimental.pallas import tpu as pltpu
from jax.experimental.pallas import tpu_sc as plsc

D = 128
SCALE = math.sqrt(128.0)
LANES = 16
NIN = 4
NOUT = 3
UNROLL = 12


def _sc_embed(idx3, lut, n_chunks, chunk, b_per_w):
    mesh = plsc.VectorSubcoreMesh(core_axis_name="c", subcore_axis_name="s")
    info = plsc.get_sparse_core_info()
    nc = info.num_cores
    B = idx3.shape[0] * idx3.shape[1] * idx3.shape[2]
    n_main = (n_chunks // UNROLL) * UNROLL

    @functools.partial(
        pl.kernel,
        mesh=mesh,
        out_type=jax.ShapeDtypeStruct((B, D), jnp.float32),
        scratch_types=[
            pltpu.VMEM((n_chunks, chunk), jnp.int32),
            pltpu.VMEM((chunk, D), jnp.float32),
            pltpu.VMEM((chunk, D), jnp.float32),
            pltpu.VMEM((chunk, D), jnp.float32),
            pltpu.VMEM((chunk, D), jnp.float32),
            pltpu.VMEM((chunk, D), jnp.float32),
            pltpu.VMEM((chunk, D), jnp.float32),
            pltpu.VMEM((chunk, D), jnp.float32),
            pltpu.SemaphoreType.DMA,
            pltpu.SemaphoreType.DMA,
            pltpu.SemaphoreType.DMA,
            pltpu.SemaphoreType.DMA,
            pltpu.SemaphoreType.DMA,
            pltpu.SemaphoreType.DMA,
            pltpu.SemaphoreType.DMA,
        ],
    )
    def k(idx_hbm, lut_hbm, out_hbm, idx_v,
          in0, in1, in2, in3, ot0, ot1, ot2,
          gs0, gs1, gs2, gs3, os0, os1, os2):
        wid = lax.axis_index("s") * nc + lax.axis_index("c")
        pltpu.sync_copy(idx_hbm.at[wid], idx_v)
        base = wid * b_per_w
        ins = (in0, in1, in2, in3)
        outs = (ot0, ot1, ot2)
        gsems = (gs0, gs1, gs2, gs3)
        osems = (os0, os1, os2)

        for b in range(NIN):
            pltpu.async_copy(lut_hbm.at[idx_v.at[b]], ins[b], gsems[b])

        def step(c, bi, bo):
            pltpu.make_async_copy(lut_hbm.at[idx_v.at[c]], ins[bi], gsems[bi]).wait()

            @pl.when(c >= NOUT)
            def _():
                pltpu.make_async_copy(
                    outs[bo], out_hbm.at[pl.ds(base, chunk)], osems[bo]
                ).wait()

            def row_body(r, carry2):
                for j in range(D // LANES):
                    sl = pl.ds(j * LANES, LANES)
                    outs[bo][r, sl] = ins[bi][r, sl] * SCALE
                return carry2

            lax.fori_loop(0, chunk, row_body, 0)

            @pl.when(c + NIN < n_chunks)
            def _():
                pltpu.async_copy(lut_hbm.at[idx_v.at[c + NIN]], ins[bi], gsems[bi])

            pltpu.async_copy(
                outs[bo], out_hbm.at[pl.ds(base + c * chunk, chunk)], osems[bo]
            )

        def outer(g, carry):
            for kk in range(UNROLL):
                c = g * UNROLL + kk
                step(c, kk % NIN, kk % NOUT)
            return carry

        lax.fori_loop(0, n_main // UNROLL, outer, 0)

        for c in range(n_main, n_chunks):
            step(c, (c % UNROLL) % NIN, (c % UNROLL) % NOUT)

        for bo in range(NOUT):
            pltpu.make_async_copy(
                outs[bo], out_hbm.at[pl.ds(base, chunk)], osems[bo]
            ).wait()

    return k(idx3, lut)


def kernel(x, lut):
    n_rows, n_cols = x.shape
    B = n_rows * n_cols
    nw = 32
    chunk = 128
    b_per_w = B // nw
    n_chunks = b_per_w // chunk
    idx3 = x.T.reshape(nw, n_chunks, chunk).astype(jnp.int32)
    out = _sc_embed(idx3, lut, n_chunks, chunk, b_per_w)
    return out.reshape(n_cols, n_rows, D).transpose(1, 0, 2)


# speedup vs baseline: 1.0010x; 1.0010x over previous
import functools
import math

import jax
import jax.numpy as jnp
from jax import lax
from jax.experimental import pallas as pl
from jax.experimental.pallas import tpu as pltpu
from jax.experimental.pallas import tpu_sc as plsc

D = 128
SCALE = math.sqrt(128.0)
LANES = 16
NIN = 3
NOUT = 3
UNROLL = 3


def _sc_embed(idx3, lut, n_chunks, chunk, b_per_w):
    mesh = plsc.VectorSubcoreMesh(core_axis_name="c", subcore_axis_name="s")
    info = plsc.get_sparse_core_info()
    nc = info.num_cores
    B = idx3.shape[0] * idx3.shape[1] * idx3.shape[2]
    n_main = (n_chunks // UNROLL) * UNROLL

    @functools.partial(
        pl.kernel,
        mesh=mesh,
        out_type=jax.ShapeDtypeStruct((B, D), jnp.float32),
        scratch_types=[
            pltpu.VMEM((n_chunks, chunk), jnp.int32),
            pltpu.VMEM((chunk, D), jnp.float32),
            pltpu.VMEM((chunk, D), jnp.float32),
            pltpu.VMEM((chunk, D), jnp.float32),
            pltpu.VMEM((chunk, D), jnp.float32),
            pltpu.VMEM((chunk, D), jnp.float32),
            pltpu.VMEM((chunk, D), jnp.float32),
            pltpu.SemaphoreType.DMA,
            pltpu.SemaphoreType.DMA,
            pltpu.SemaphoreType.DMA,
            pltpu.SemaphoreType.DMA,
            pltpu.SemaphoreType.DMA,
            pltpu.SemaphoreType.DMA,
        ],
    )
    def k(idx_hbm, lut_hbm, out_hbm, idx_v,
          in0, in1, in2, ot0, ot1, ot2,
          gs0, gs1, gs2, os0, os1, os2):
        wid = lax.axis_index("s") * nc + lax.axis_index("c")
        pltpu.sync_copy(idx_hbm.at[wid], idx_v)
        base = wid * b_per_w
        ins = (in0, in1, in2)
        outs = (ot0, ot1, ot2)
        gsems = (gs0, gs1, gs2)
        osems = (os0, os1, os2)

        for b in range(NIN):
            pltpu.async_copy(lut_hbm.at[idx_v.at[b]], ins[b], gsems[b])

        def step(c, bi, bo):
            pltpu.make_async_copy(lut_hbm.at[idx_v.at[c]], ins[bi], gsems[bi]).wait()

            @pl.when(c >= NOUT)
            def _():
                pltpu.make_async_copy(
                    outs[bo], out_hbm.at[pl.ds(base, chunk)], osems[bo]
                ).wait()

            def row_body(r, carry2):
                for j in range(D // LANES):
                    sl = pl.ds(j * LANES, LANES)
                    outs[bo][r, sl] = ins[bi][r, sl] * SCALE
                return carry2

            lax.fori_loop(0, chunk, row_body, 0)

            @pl.when(c + NIN < n_chunks)
            def _():
                pltpu.async_copy(lut_hbm.at[idx_v.at[c + NIN]], ins[bi], gsems[bi])

            pltpu.async_copy(
                outs[bo], out_hbm.at[pl.ds(base + c * chunk, chunk)], osems[bo]
            )

        def outer(g, carry):
            for kk in range(UNROLL):
                c = g * UNROLL + kk
                step(c, kk % NIN, kk % NOUT)
            return carry

        lax.fori_loop(0, n_main // UNROLL, outer, 0)

        for c in range(n_main, n_chunks):
            step(c, (c % UNROLL) % NIN, (c % UNROLL) % NOUT)

        for bo in range(NOUT):
            pltpu.make_async_copy(
                outs[bo], out_hbm.at[pl.ds(base, chunk)], osems[bo]
            ).wait()

    return k(idx3, lut)


def kernel(x, lut):
    n_rows, n_cols = x.shape
    B = n_rows * n_cols
    nw = 32
    chunk = 128
    b_per_w = B // nw
    n_chunks = b_per_w // chunk
    idx3 = x.T.reshape(nw, n_chunks, chunk).astype(jnp.int32)
    out = _sc_embed(idx3, lut, n_chunks, chunk, b_per_w)
    return out.reshape(n_cols, n_rows, D).transpose(1, 0, 2)
